# Initial kernel scaffold; baseline (speedup 1.0000x reference)
import jax
import jax.numpy as jnp
from jax import lax
from jax.experimental import pallas as pl
from jax.experimental.pallas import tpu as pltpu
from jax.experimental.pallas import tpu_sc as plsc

N = 50000
E = 800000
H = 64
ITERS = 3

NC = 2
NS = 16
LN = 16

GC = 128
NCHUNK = -(-E // GC)
GPT = -(-NCHUNK // (NC * NS))
CPAD = GPT * NC * NS
EPAD = CPAD * GC

SCH = NCHUNK // NS
SEXTRA = NCHUNK - SCH * NS
SIDX = SCH + 1
NPT = N // NS
HC = H // NC

BN = 2000
BE = 2000


def _ln_act(h):
    m = jnp.mean(h, axis=-1, keepdims=True)
    v = jnp.mean((h - m) * (h - m), axis=-1, keepdims=True)
    h = (h - m) * lax.rsqrt(v + 1e-5)
    return h * jax.nn.sigmoid(h)


def _dot(a, b):
    return jnp.dot(a, b, preferred_element_type=jnp.float32)


def _enc_body(z_ref, w1_ref, b1_ref, w2_ref, b2_ref, wea_ref, web_ref,
              wu0_ref, wv0_ref, x_ref, ue_ref, ve_ref, u0_ref, v0_ref):
    h = z_ref[...] * w1_ref[...] + b1_ref[...]
    h = _ln_act(h)
    x = jnp.tanh(_dot(h, w2_ref[...]) + b2_ref[...])
    x_ref[...] = x
    ue_ref[...] = _dot(x, wea_ref[...])
    ve_ref[...] = _dot(x, web_ref[...])
    u0_ref[...] = _dot(x, wu0_ref[...])
    v0_ref[...] = _dot(x, wv0_ref[...])


def _edge0_body(g_ref, b1_ref, w2_ref, b2_ref, e_ref):
    h = _ln_act(g_ref[...] + b1_ref[...])
    e_ref[...] = jnp.tanh(_dot(h, w2_ref[...]) + b2_ref[...])


def _edge_body(e_ref, e0_ref, g_ref, wa_ref, wb_ref, b1_ref, w2_ref, b2_ref,
               out_ref):
    h = _dot(e_ref[...], wa_ref[...]) + _dot(e0_ref[...], wb_ref[...])
    h = _ln_act(h + g_ref[...] + b1_ref[...])
    out_ref[...] = jnp.tanh(_dot(h, w2_ref[...]) + b2_ref[...])


def _node_body(ms_ref, md_ref, x_ref, x0_ref, wm1_ref, wm2_ref, wx1_ref,
               wx2_ref, b1_ref, w2_ref, b2_ref, xo_ref):
    h = (_dot(ms_ref[...], wm1_ref[...]) + _dot(md_ref[...], wm2_ref[...]) +
         _dot(x_ref[...], wx1_ref[...]) + _dot(x0_ref[...], wx2_ref[...]))
    h = _ln_act(h + b1_ref[...])
    xo_ref[...] = jnp.tanh(_dot(h, w2_ref[...]) + b2_ref[...])


def _node_tab_body(ms_ref, md_ref, x_ref, x0_ref, wm1_ref, wm2_ref, wx1_ref,
                   wx2_ref, b1_ref, w2_ref, b2_ref, wua_ref, wub_ref,
                   wva_ref, wvb_ref, xo_ref, u_ref, v_ref):
    h = (_dot(ms_ref[...], wm1_ref[...]) + _dot(md_ref[...], wm2_ref[...]) +
         _dot(x_ref[...], wx1_ref[...]) + _dot(x0_ref[...], wx2_ref[...]))
    h = _ln_act(h + b1_ref[...])
    x = jnp.tanh(_dot(h, w2_ref[...]) + b2_ref[...])
    xo_ref[...] = x
    x0v = x0_ref[...]
    u_ref[...] = _dot(x, wua_ref[...]) + _dot(x0v, wub_ref[...])
    v_ref[...] = _dot(x, wva_ref[...]) + _dot(x0v, wvb_ref[...])


def _decode_body(e_ref, wd1_ref, bd1_ref, wd2_ref, bd2_ref, wo1_ref, bo1_ref,
                 wo2_ref, bo2_ref, out_ref):
    h = _ln_act(_dot(e_ref[...], wd1_ref[...]) + bd1_ref[...])
    hd = jnp.tanh(_dot(h, wd2_ref[...]) + bd2_ref[...])
    h = _ln_act(_dot(hd, wo1_ref[...]) + bo1_ref[...])
    out_ref[...] = (jnp.sum(h * wo2_ref[...], axis=-1, keepdims=True)
                    + bo2_ref[...])


def _row_spec(b, w):
    return pl.BlockSpec((b, w), lambda i: (i, 0))


def _full_spec(shape):
    return pl.BlockSpec(shape, lambda i: tuple(0 for _ in shape))


def _tc_call(body, n_rows, block, in_widths, out_widths):
    grid = (n_rows // block,)
    in_specs = [
        _row_spec(block, w) if isinstance(w, int) else _full_spec(w)
        for w in in_widths
    ]
    out_specs = [_row_spec(block, w) for w in out_widths]
    out_shape = [jax.ShapeDtypeStruct((n_rows, w), jnp.float32)
                 for w in out_widths]
    if len(out_widths) == 1:
        out_specs, out_shape = out_specs[0], out_shape[0]
    return pl.pallas_call(
        body, grid=grid, in_specs=in_specs, out_specs=out_specs,
        out_shape=out_shape)


def _gather_body(u_hbm, v_hbm, s_hbm, d_hbm, g_hbm,
                 sidx, didx, ubuf, vbuf, sem_u, sem_v):
    c = lax.axis_index("c")
    s = lax.axis_index("s")
    wid = s * NC + c
    base = wid * GPT
    pltpu.sync_copy(s_hbm.at[pl.ds(base, GPT)], sidx)
    pltpu.sync_copy(d_hbm.at[pl.ds(base, GPT)], didx)

    def chunk(j, carry):
        cp_u = pltpu.async_copy(u_hbm.at[sidx.at[j]], ubuf, sem_u)
        cp_v = pltpu.async_copy(v_hbm.at[didx.at[j]], vbuf, sem_v)
        cp_u.wait()
        cp_v.wait()

        def addrow(r, cc):
            for t in range(H // LN):
                sl = pl.ds(t * LN, LN)
                plsc.addupdate(ubuf.at[r, sl], vbuf[r, sl])
            return cc

        lax.fori_loop(0, GC, addrow, 0)
        pltpu.sync_copy(ubuf, g_hbm.at[pl.ds((base + j) * GC, GC)])
        return carry

    lax.fori_loop(0, GPT, chunk, 0)


def _make_gather():
    mesh = plsc.VectorSubcoreMesh(core_axis_name="c", subcore_axis_name="s",
                                  num_cores=NC, num_subcores=NS)
    return pl.kernel(
        _gather_body,
        out_type=jax.ShapeDtypeStruct((EPAD, H), jnp.float32),
        mesh=mesh,
        scratch_types=[
            pltpu.VMEM((GPT, GC), jnp.int32),
            pltpu.VMEM((GPT, GC), jnp.int32),
            pltpu.VMEM((GC, H), jnp.float32),
            pltpu.VMEM((GC, H), jnp.float32),
            pltpu.SemaphoreType.DMA,
            pltpu.SemaphoreType.DMA,
        ],
    )


def _scatter_body(e_hbm, d_hbm, s_hbm, z_hbm, ms_hbm, md_hbm,
                  idxv, valbuf, acc):
    c = lax.axis_index("c")
    s = lax.axis_index("s")
    col = c * HC
    base = s * SCH + jnp.minimum(s, SEXTRA)
    nch = SCH + jnp.where(s < SEXTRA, 1, 0)
    nslc = pl.ds(s * NPT, NPT)

    for i_hbm, o_hbm in ((d_hbm, ms_hbm), (s_hbm, md_hbm)):
        pltpu.sync_copy(z_hbm, acc.at[nslc])
        pltpu.sync_copy(i_hbm.at[pl.ds(base, SIDX)], idxv)
        plsc.subcore_barrier()

        def chunk(jj, carry):
            q = base + jj
            pltpu.sync_copy(e_hbm.at[pl.ds(q * GC, GC), pl.ds(col, HC)],
                            valbuf)
            pltpu.sync_copy(valbuf, acc.at[idxv.at[jj]], add=True)
            return carry

        lax.fori_loop(0, nch, chunk, 0)
        plsc.subcore_barrier()
        pltpu.sync_copy(acc.at[nslc], o_hbm.at[nslc, pl.ds(col, HC)])
        plsc.subcore_barrier()


def _make_scatter():
    mesh = plsc.VectorSubcoreMesh(core_axis_name="c", subcore_axis_name="s",
                                  num_cores=NC, num_subcores=NS)
    return pl.kernel(
        _scatter_body,
        out_type=[jax.ShapeDtypeStruct((N, H), jnp.float32),
                  jax.ShapeDtypeStruct((N, H), jnp.float32)],
        mesh=mesh,
        scratch_types=[
            pltpu.VMEM((SIDX, GC), jnp.int32),
            pltpu.VMEM((GC, HC), jnp.float32),
            pltpu.VMEM_SHARED((N, HC), jnp.float32),
        ],
    )


def _w(p, i):
    return p[i]["W"]


def _b(p, i):
    return p[i]["b"].reshape(1, -1)


@jax.jit
def kernel(z, params, edge_index):
    ei = edge_index.astype(jnp.int32)
    src = ei[0]
    dst = ei[1]
    srcr = jnp.pad(src.reshape(NCHUNK, GC), ((0, CPAD - NCHUNK), (0, 0)))
    dstr = jnp.pad(dst.reshape(NCHUNK, GC), ((0, CPAD - NCHUNK), (0, 0)))
    zeros_tile = jnp.zeros((NPT, HC), jnp.float32)

    ne = params["node_encoder"]
    ee = params["edge_encoder"]
    en = params["edge_network"]
    nn = params["node_network"]
    dec = params["edge_decoder"]
    eot = params["edge_output_transform"]

    enW = [_w(en[i], 0) for i in range(ITERS)]
    nnW = [_w(nn[i], 0) for i in range(ITERS)]

    eeW1 = _w(ee, 0)
    wu0 = enW[0][2 * H:3 * H] + enW[0][3 * H:4 * H]
    wv0 = enW[0][4 * H:5 * H] + enW[0][5 * H:6 * H]

    enc = _tc_call(_enc_body, N, BN,
                   [1, (1, H), (1, H), (H, H), (1, H), (H, H), (H, H),
                    (H, H), (H, H)],
                   [H, H, H, H, H])
    x0, ue, ve, u0, v0 = enc(z.reshape(N, 1), _w(ne, 0), _b(ne, 0),
                             _w(ne, 1), _b(ne, 1), eeW1[:H], eeW1[H:],
                             wu0, wv0)

    gather = _make_gather()
    scatter = _make_scatter()

    edge0 = _tc_call(_edge0_body, E, BE,
                     [H, (1, H), (H, H), (1, H)], [H])
    edge_mlp = _tc_call(_edge_body, E, BE,
                        [H, H, H, (H, H), (H, H), (1, H), (H, H), (1, H)],
                        [H])
    node_tab = _tc_call(_node_tab_body, N, BN,
                        [H, H, H, H] + [(H, H)] * 4 + [(1, H), (H, H), (1, H)]
                        + [(H, H)] * 4,
                        [H, H, H])
    node_last = _tc_call(_node_body, N, BN,
                         [H, H, H, H] + [(H, H)] * 4
                         + [(1, H), (H, H), (1, H)],
                         [H])

    g = gather(ue, ve, srcr, dstr)
    e0 = edge0(g[:E], _b(ee, 0), _w(ee, 1), _b(ee, 1))

    e = e0
    x = x0
    u, v = u0, v0
    for i in range(ITERS):
        g = gather(u, v, srcr, dstr)
        e = edge_mlp(e, e0, g[:E], enW[i][:H], enW[i][H:2 * H], _b(en[i], 0),
                     _w(en[i], 1), _b(en[i], 1))
        ms, md = scatter(e, dstr, srcr, zeros_tile)
        if i < ITERS - 1:
            x, u, v = node_tab(ms, md, x, x0, nnW[i][:H], nnW[i][H:2 * H],
                               nnW[i][2 * H:3 * H], nnW[i][3 * H:],
                               _b(nn[i], 0), _w(nn[i], 1), _b(nn[i], 1),
                               enW[i + 1][2 * H:3 * H],
                               enW[i + 1][3 * H:4 * H],
                               enW[i + 1][4 * H:5 * H],
                               enW[i + 1][5 * H:6 * H])
        else:
            x = node_last(ms, md, x, x0, nnW[i][:H], nnW[i][H:2 * H],
                          nnW[i][2 * H:3 * H], nnW[i][3 * H:],
                          _b(nn[i], 0), _w(nn[i], 1), _b(nn[i], 1))

    decode = _tc_call(_decode_body, E, BE,
                      [H, (H, H), (1, H), (H, H), (1, H), (H, H), (1, H),
                       (1, H), (1, 1)],
                      [1])
    out = decode(e, _w(dec, 0), _b(dec, 0), _w(dec, 1), _b(dec, 1),
                 _w(eot, 0), _b(eot, 0), _w(eot, 1).reshape(1, H),
                 _b(eot, 1).reshape(1, 1))
    return out[:, 0]


# speedup vs baseline: 1.3415x; 1.3415x over previous
import jax
import jax.numpy as jnp
from jax import lax
from jax.experimental import pallas as pl
from jax.experimental.pallas import tpu as pltpu
from jax.experimental.pallas import tpu_sc as plsc

N = 50000
E = 800000
H = 64
ITERS = 3

NC = 2
NS = 16
LN = 16

GC = 128
NCHUNK = E // GC
GPT = 200
CPAD = GPT * NC * NS
EPAD = CPAD * GC

SCH = 392
SLAST = NCHUNK - SCH * (NS - 1)
SB = 56
NPT = 3128
NPTL = N - NPT * (NS - 1)
HC = H // NC

BN = 2000
BE = 2000


def _ln_act(h):
    m = jnp.mean(h, axis=-1, keepdims=True)
    v = jnp.mean((h - m) * (h - m), axis=-1, keepdims=True)
    h = (h - m) * lax.rsqrt(v + 1e-5)
    return h * jax.nn.sigmoid(h)


def _dot(a, b):
    return jnp.dot(a, b, preferred_element_type=jnp.float32,
                   precision=lax.Precision.HIGHEST)


def _enc_body(z_ref, w1_ref, b1_ref, w2_ref, b2_ref, wea_ref, web_ref,
              wu0_ref, wv0_ref, x_ref, ue_ref, ve_ref, u0_ref, v0_ref):
    h = z_ref[...] * w1_ref[...] + b1_ref[...]
    h = _ln_act(h)
    x = jnp.tanh(_dot(h, w2_ref[...]) + b2_ref[...])
    x_ref[...] = x
    ue_ref[...] = _dot(x, wea_ref[...])
    ve_ref[...] = _dot(x, web_ref[...])
    u0_ref[...] = _dot(x, wu0_ref[...])
    v0_ref[...] = _dot(x, wv0_ref[...])


def _edge0_body(g_ref, b1_ref, w2_ref, b2_ref, e_ref):
    h = _ln_act(g_ref[...] + b1_ref[...])
    e_ref[...] = jnp.tanh(_dot(h, w2_ref[...]) + b2_ref[...])


def _edge_body(e_ref, e0_ref, g_ref, wa_ref, wb_ref, b1_ref, w2_ref, b2_ref,
               out_ref):
    h = _dot(e_ref[...], wa_ref[...]) + _dot(e0_ref[...], wb_ref[...])
    h = _ln_act(h + g_ref[...] + b1_ref[...])
    out_ref[...] = jnp.tanh(_dot(h, w2_ref[...]) + b2_ref[...])


def _node_body(ms_ref, md_ref, x_ref, x0_ref, wm1_ref, wm2_ref, wx1_ref,
               wx2_ref, b1_ref, w2_ref, b2_ref, xo_ref):
    h = (_dot(ms_ref[...], wm1_ref[...]) + _dot(md_ref[...], wm2_ref[...]) +
         _dot(x_ref[...], wx1_ref[...]) + _dot(x0_ref[...], wx2_ref[...]))
    h = _ln_act(h + b1_ref[...])
    xo_ref[...] = jnp.tanh(_dot(h, w2_ref[...]) + b2_ref[...])


def _node_tab_body(ms_ref, md_ref, x_ref, x0_ref, wm1_ref, wm2_ref, wx1_ref,
                   wx2_ref, b1_ref, w2_ref, b2_ref, wua_ref, wub_ref,
                   wva_ref, wvb_ref, xo_ref, u_ref, v_ref):
    h = (_dot(ms_ref[...], wm1_ref[...]) + _dot(md_ref[...], wm2_ref[...]) +
         _dot(x_ref[...], wx1_ref[...]) + _dot(x0_ref[...], wx2_ref[...]))
    h = _ln_act(h + b1_ref[...])
    x = jnp.tanh(_dot(h, w2_ref[...]) + b2_ref[...])
    xo_ref[...] = x
    x0v = x0_ref[...]
    u_ref[...] = _dot(x, wua_ref[...]) + _dot(x0v, wub_ref[...])
    v_ref[...] = _dot(x, wva_ref[...]) + _dot(x0v, wvb_ref[...])


def _decode_body(e_ref, wd1_ref, bd1_ref, wd2_ref, bd2_ref, wo1_ref, bo1_ref,
                 wo2_ref, bo2_ref, out_ref):
    h = _ln_act(_dot(e_ref[...], wd1_ref[...]) + bd1_ref[...])
    hd = jnp.tanh(_dot(h, wd2_ref[...]) + bd2_ref[...])
    h = _ln_act(_dot(hd, wo1_ref[...]) + bo1_ref[...])
    out_ref[...] = (jnp.sum(h * wo2_ref[...], axis=-1, keepdims=True)
                    + bo2_ref[...])


def _row_spec(b, w):
    return pl.BlockSpec((b, w), lambda i: (i, 0))


def _full_spec(shape):
    return pl.BlockSpec(shape, lambda i: tuple(0 for _ in shape))


def _tc_call(body, n_rows, block, in_widths, out_widths):
    grid = (n_rows // block,)
    in_specs = [
        _row_spec(block, w) if isinstance(w, int) else _full_spec(w)
        for w in in_widths
    ]
    out_specs = [_row_spec(block, w) for w in out_widths]
    out_shape = [jax.ShapeDtypeStruct((n_rows, w), jnp.float32)
                 for w in out_widths]
    if len(out_widths) == 1:
        out_specs, out_shape = out_specs[0], out_shape[0]
    return pl.pallas_call(
        body, grid=grid, in_specs=in_specs, out_specs=out_specs,
        out_shape=out_shape)


def _gather_body(u_hbm, v_hbm, s_hbm, d_hbm, g_hbm,
                 sidx, didx, ubuf, vbuf, sem_u, sem_v):
    c = lax.axis_index("c")
    s = lax.axis_index("s")
    wid = s * NC + c
    base = wid * GPT
    pltpu.sync_copy(s_hbm.at[pl.ds(base, GPT)], sidx)
    pltpu.sync_copy(d_hbm.at[pl.ds(base, GPT)], didx)

    def chunk(j, carry):
        cp_u = pltpu.async_copy(u_hbm.at[sidx.at[j]], ubuf, sem_u)
        cp_v = pltpu.async_copy(v_hbm.at[didx.at[j]], vbuf, sem_v)
        cp_u.wait()
        cp_v.wait()

        def addrow(r, cc):
            for t in range(H // LN):
                sl = pl.ds(t * LN, LN)
                plsc.addupdate(ubuf.at[r, sl], vbuf[r, sl])
            return cc

        lax.fori_loop(0, GC, addrow, 0)
        pltpu.sync_copy(ubuf, g_hbm.at[pl.ds((base + j) * GC, GC)])
        return carry

    lax.fori_loop(0, GPT, chunk, 0)


def _make_gather():
    mesh = plsc.VectorSubcoreMesh(core_axis_name="c", subcore_axis_name="s",
                                  num_cores=NC, num_subcores=NS)
    return pl.kernel(
        _gather_body,
        out_type=jax.ShapeDtypeStruct((EPAD, H), jnp.float32),
        mesh=mesh,
        compiler_params=pltpu.CompilerParams(use_tc_tiling_on_sc=False),
        scratch_types=[
            pltpu.VMEM((GPT, GC), jnp.int32),
            pltpu.VMEM((GPT, GC), jnp.int32),
            pltpu.VMEM((GC, H), jnp.float32),
            pltpu.VMEM((GC, H), jnp.float32),
            pltpu.SemaphoreType.DMA,
            pltpu.SemaphoreType.DMA,
        ],
    )


def _scatter_body(e_hbm, d_hbm, s_hbm, z_hbm, ms_hbm, md_hbm,
                  idxv, valbuf, acc):
    c = lax.axis_index("c")
    s = lax.axis_index("s")
    col = c * HC
    base = s * SCH
    nch = jnp.where(s < NS - 1, SCH, SLAST)
    nbase = s * NPT

    for i_hbm, o_hbm in ((d_hbm, ms_hbm), (s_hbm, md_hbm)):
        @pl.when(s < NS - 1)
        def _():
            pltpu.sync_copy(z_hbm, acc.at[pl.ds(nbase, NPT)])

        @pl.when(s == NS - 1)
        def _():
            pltpu.sync_copy(z_hbm.at[pl.ds(0, NPTL)],
                            acc.at[pl.ds(nbase, NPTL)])

        plsc.subcore_barrier()

        def blk(b, carry):
            pltpu.sync_copy(i_hbm.at[pl.ds(base + b * SB, SB)], idxv)
            nin = jnp.minimum(SB, nch - b * SB)

            def chunk(jj, carry2):
                q = base + b * SB + jj
                pltpu.sync_copy(e_hbm.at[pl.ds(q * GC, GC), pl.ds(col, HC)],
                                valbuf)
                pltpu.sync_copy(valbuf, acc.at[idxv.at[jj]], add=True)
                return carry2

            lax.fori_loop(0, nin, chunk, 0)
            return carry

        lax.fori_loop(0, (SCH + SB - 1) // SB, blk, 0)
        plsc.subcore_barrier()

        @pl.when(s < NS - 1)
        def _():
            pltpu.sync_copy(acc.at[pl.ds(nbase, NPT)],
                            o_hbm.at[pl.ds(nbase, NPT), pl.ds(col, HC)])

        @pl.when(s == NS - 1)
        def _():
            pltpu.sync_copy(acc.at[pl.ds(nbase, NPTL)],
                            o_hbm.at[pl.ds(nbase, NPTL), pl.ds(col, HC)])

        plsc.subcore_barrier()


def _make_scatter():
    mesh = plsc.VectorSubcoreMesh(core_axis_name="c", subcore_axis_name="s",
                                  num_cores=NC, num_subcores=NS)
    return pl.kernel(
        _scatter_body,
        out_type=[jax.ShapeDtypeStruct((N, H), jnp.float32),
                  jax.ShapeDtypeStruct((N, H), jnp.float32)],
        mesh=mesh,
        compiler_params=pltpu.CompilerParams(use_tc_tiling_on_sc=False),
        scratch_types=[
            pltpu.VMEM((SB, GC), jnp.int32),
            pltpu.VMEM((GC, HC), jnp.float32),
            pltpu.VMEM_SHARED((N, HC), jnp.float32),
        ],
    )


def _w(p, i):
    return p[i]["W"]


def _b(p, i):
    return p[i]["b"].reshape(1, -1)


@jax.jit
def kernel(z, params, edge_index):
    ei = edge_index.astype(jnp.int32)
    src = ei[0]
    dst = ei[1]
    srcr = jnp.pad(src.reshape(NCHUNK, GC), ((0, CPAD - NCHUNK), (0, 0)))
    dstr = jnp.pad(dst.reshape(NCHUNK, GC), ((0, CPAD - NCHUNK), (0, 0)))
    zeros_tile = jnp.zeros((NPT, HC), jnp.float32)

    ne = params["node_encoder"]
    ee = params["edge_encoder"]
    en = params["edge_network"]
    nn = params["node_network"]
    dec = params["edge_decoder"]
    eot = params["edge_output_transform"]

    enW = [_w(en[i], 0) for i in range(ITERS)]
    nnW = [_w(nn[i], 0) for i in range(ITERS)]

    eeW1 = _w(ee, 0)
    wu0 = enW[0][2 * H:3 * H] + enW[0][3 * H:4 * H]
    wv0 = enW[0][4 * H:5 * H] + enW[0][5 * H:6 * H]

    enc = _tc_call(_enc_body, N, BN,
                   [1, (1, H), (1, H), (H, H), (1, H), (H, H), (H, H),
                    (H, H), (H, H)],
                   [H, H, H, H, H])
    x0, ue, ve, u0, v0 = enc(z.reshape(N, 1), _w(ne, 0), _b(ne, 0),
                             _w(ne, 1), _b(ne, 1), eeW1[:H], eeW1[H:],
                             wu0, wv0)

    gather = _make_gather()
    scatter = _make_scatter()

    edge0 = _tc_call(_edge0_body, E, BE,
                     [H, (1, H), (H, H), (1, H)], [H])
    edge_mlp = _tc_call(_edge_body, E, BE,
                        [H, H, H, (H, H), (H, H), (1, H), (H, H), (1, H)],
                        [H])
    node_tab = _tc_call(_node_tab_body, N, BN,
                        [H, H, H, H] + [(H, H)] * 4 + [(1, H), (H, H), (1, H)]
                        + [(H, H)] * 4,
                        [H, H, H])
    node_last = _tc_call(_node_body, N, BN,
                         [H, H, H, H] + [(H, H)] * 4
                         + [(1, H), (H, H), (1, H)],
                         [H])

    g = gather(ue, ve, srcr, dstr)
    e0 = edge0(g, _b(ee, 0), _w(ee, 1), _b(ee, 1))

    e = e0
    x = x0
    u, v = u0, v0
    for i in range(ITERS):
        g = gather(u, v, srcr, dstr)
        e = edge_mlp(e, e0, g, enW[i][:H], enW[i][H:2 * H], _b(en[i], 0),
                     _w(en[i], 1), _b(en[i], 1))
        ms, md = scatter(e, dstr, srcr, zeros_tile)
        if i < ITERS - 1:
            x, u, v = node_tab(ms, md, x, x0, nnW[i][:H], nnW[i][H:2 * H],
                               nnW[i][2 * H:3 * H], nnW[i][3 * H:],
                               _b(nn[i], 0), _w(nn[i], 1), _b(nn[i], 1),
                               enW[i + 1][2 * H:3 * H],
                               enW[i + 1][3 * H:4 * H],
                               enW[i + 1][4 * H:5 * H],
                               enW[i + 1][5 * H:6 * H])
        else:
            x = node_last(ms, md, x, x0, nnW[i][:H], nnW[i][H:2 * H],
                          nnW[i][2 * H:3 * H], nnW[i][3 * H:],
                          _b(nn[i], 0), _w(nn[i], 1), _b(nn[i], 1))

    decode = _tc_call(_decode_body, E, BE,
                      [H, (H, H), (1, H), (H, H), (1, H), (H, H), (1, H),
                       (1, H), (1, 1)],
                      [1])
    out = decode(e, _w(dec, 0), _b(dec, 0), _w(dec, 1), _b(dec, 1),
                 _w(eot, 0), _b(eot, 0), _w(eot, 1).reshape(1, H),
                 _b(eot, 1).reshape(1, 1))
    return out[:, 0]


# speedup vs baseline: 1.4722x; 1.0974x over previous
import jax
import jax.numpy as jnp
from jax import lax
from jax.experimental import pallas as pl
from jax.experimental.pallas import tpu as pltpu
from jax.experimental.pallas import tpu_sc as plsc

N = 50000
E = 800000
H = 64
ITERS = 3

NC = 2
NS = 16
LN = 16

GC = 128
NCHUNK = E // GC
GPT = 200
CPAD = GPT * NC * NS
EPAD = CPAD * GC

SCH = 392
SLAST = NCHUNK - SCH * (NS - 1)
SB = 56
NPT = 3128
NPTL = N - NPT * (NS - 1)
HC = H // NC

BN = 2000
BE = 2000


def _ln_act(h):
    m = jnp.mean(h, axis=-1, keepdims=True)
    v = jnp.mean((h - m) * (h - m), axis=-1, keepdims=True)
    h = (h - m) * lax.rsqrt(v + 1e-5)
    return h * jax.nn.sigmoid(h)


def _dot(a, b):
    return jnp.dot(a, b, preferred_element_type=jnp.float32,
                   precision=lax.Precision.HIGHEST)


def _enc_body(z_ref, w1_ref, b1_ref, w2_ref, b2_ref, wea_ref, web_ref,
              wu0_ref, wv0_ref, x_ref, ue_ref, ve_ref, u0_ref, v0_ref):
    h = z_ref[...] * w1_ref[...] + b1_ref[...]
    h = _ln_act(h)
    x = jnp.tanh(_dot(h, w2_ref[...]) + b2_ref[...])
    x_ref[...] = x
    ue_ref[...] = _dot(x, wea_ref[...])
    ve_ref[...] = _dot(x, web_ref[...])
    u0_ref[...] = _dot(x, wu0_ref[...])
    v0_ref[...] = _dot(x, wv0_ref[...])


def _edge0_body(g_ref, b1_ref, w2_ref, b2_ref, e_ref):
    h = _ln_act(g_ref[...] + b1_ref[...])
    e_ref[...] = jnp.tanh(_dot(h, w2_ref[...]) + b2_ref[...])


def _edge_body(e_ref, e0_ref, g_ref, wa_ref, wb_ref, b1_ref, w2_ref, b2_ref,
               out_ref):
    h = _dot(e_ref[...], wa_ref[...]) + _dot(e0_ref[...], wb_ref[...])
    h = _ln_act(h + g_ref[...] + b1_ref[...])
    out_ref[...] = jnp.tanh(_dot(h, w2_ref[...]) + b2_ref[...])


def _node_body(ms_ref, md_ref, x_ref, x0_ref, wm1_ref, wm2_ref, wx1_ref,
               wx2_ref, b1_ref, w2_ref, b2_ref, xo_ref):
    h = (_dot(ms_ref[...], wm1_ref[...]) + _dot(md_ref[...], wm2_ref[...]) +
         _dot(x_ref[...], wx1_ref[...]) + _dot(x0_ref[...], wx2_ref[...]))
    h = _ln_act(h + b1_ref[...])
    xo_ref[...] = jnp.tanh(_dot(h, w2_ref[...]) + b2_ref[...])


def _node_tab_body(ms_ref, md_ref, x_ref, x0_ref, wm1_ref, wm2_ref, wx1_ref,
                   wx2_ref, b1_ref, w2_ref, b2_ref, wua_ref, wub_ref,
                   wva_ref, wvb_ref, xo_ref, u_ref, v_ref):
    h = (_dot(ms_ref[...], wm1_ref[...]) + _dot(md_ref[...], wm2_ref[...]) +
         _dot(x_ref[...], wx1_ref[...]) + _dot(x0_ref[...], wx2_ref[...]))
    h = _ln_act(h + b1_ref[...])
    x = jnp.tanh(_dot(h, w2_ref[...]) + b2_ref[...])
    xo_ref[...] = x
    x0v = x0_ref[...]
    u_ref[...] = _dot(x, wua_ref[...]) + _dot(x0v, wub_ref[...])
    v_ref[...] = _dot(x, wva_ref[...]) + _dot(x0v, wvb_ref[...])


def _decode_body(e_ref, wd1_ref, bd1_ref, wd2_ref, bd2_ref, wo1_ref, bo1_ref,
                 wo2_ref, bo2_ref, out_ref):
    h = _ln_act(_dot(e_ref[...], wd1_ref[...]) + bd1_ref[...])
    hd = jnp.tanh(_dot(h, wd2_ref[...]) + bd2_ref[...])
    h = _ln_act(_dot(hd, wo1_ref[...]) + bo1_ref[...])
    out_ref[...] = (jnp.sum(h * wo2_ref[...], axis=-1, keepdims=True)
                    + bo2_ref[...])


def _row_spec(b, w):
    return pl.BlockSpec((b, w), lambda i: (i, 0))


def _full_spec(shape):
    return pl.BlockSpec(shape, lambda i: tuple(0 for _ in shape))


def _tc_call(body, n_rows, block, in_widths, out_widths):
    grid = (n_rows // block,)
    in_specs = [
        _row_spec(block, w) if isinstance(w, int) else _full_spec(w)
        for w in in_widths
    ]
    out_specs = [_row_spec(block, w) for w in out_widths]
    out_shape = [jax.ShapeDtypeStruct((n_rows, w), jnp.float32)
                 for w in out_widths]
    if len(out_widths) == 1:
        out_specs, out_shape = out_specs[0], out_shape[0]
    return pl.pallas_call(
        body, grid=grid, in_specs=in_specs, out_specs=out_specs,
        out_shape=out_shape)


def _gather_body(u_hbm, v_hbm, s_hbm, d_hbm, g_hbm,
                 sidx, didx, ub0, vb0, ub1, vb1,
                 sem_u0, sem_v0, sem_u1, sem_v1):
    c = lax.axis_index("c")
    s = lax.axis_index("s")
    wid = s * NC + c
    base = wid * GPT
    pltpu.sync_copy(s_hbm.at[pl.ds(base, GPT)], sidx)
    pltpu.sync_copy(d_hbm.at[pl.ds(base, GPT)], didx)

    def fire(j, ub, vb, su, sv):
        pltpu.async_copy(u_hbm.at[sidx.at[j]], ub, su)
        pltpu.async_copy(v_hbm.at[didx.at[j]], vb, sv)

    def wait(ub, vb, su, sv):
        pltpu.make_async_copy(u_hbm.at[pl.ds(0, GC)], ub, su).wait()
        pltpu.make_async_copy(u_hbm.at[pl.ds(0, GC)], vb, sv).wait()

    def add_rows(ub, vb):
        def addrow(r4, cc):
            for rr in range(4):
                r = r4 * 4 + rr
                for t in range(H // LN):
                    sl = pl.ds(t * LN, LN)
                    plsc.addupdate(ub.at[r, sl], vb[r, sl])
            return cc

        lax.fori_loop(0, GC // 4, addrow, 0)

    fire(0, ub0, vb0, sem_u0, sem_v0)

    def pair(p, carry):
        j0 = 2 * p
        j1 = j0 + 1
        fire(j1, ub1, vb1, sem_u1, sem_v1)
        wait(ub0, vb0, sem_u0, sem_v0)
        add_rows(ub0, vb0)
        pltpu.sync_copy(ub0, g_hbm.at[pl.ds((base + j0) * GC, GC)])

        @pl.when(p + 1 < GPT // 2)
        def _():
            fire(j0 + 2, ub0, vb0, sem_u0, sem_v0)

        wait(ub1, vb1, sem_u1, sem_v1)
        add_rows(ub1, vb1)
        pltpu.sync_copy(ub1, g_hbm.at[pl.ds((base + j1) * GC, GC)])
        return carry

    lax.fori_loop(0, GPT // 2, pair, 0)


def _make_gather():
    mesh = plsc.VectorSubcoreMesh(core_axis_name="c", subcore_axis_name="s",
                                  num_cores=NC, num_subcores=NS)
    return pl.kernel(
        _gather_body,
        out_type=jax.ShapeDtypeStruct((EPAD, H), jnp.float32),
        mesh=mesh,
        compiler_params=pltpu.CompilerParams(use_tc_tiling_on_sc=False),
        scratch_types=[
            pltpu.VMEM((GPT, GC), jnp.int32),
            pltpu.VMEM((GPT, GC), jnp.int32),
            pltpu.VMEM((GC, H), jnp.float32),
            pltpu.VMEM((GC, H), jnp.float32),
            pltpu.VMEM((GC, H), jnp.float32),
            pltpu.VMEM((GC, H), jnp.float32),
            pltpu.SemaphoreType.DMA,
            pltpu.SemaphoreType.DMA,
            pltpu.SemaphoreType.DMA,
            pltpu.SemaphoreType.DMA,
        ],
    )


def _scatter_body(e_hbm, d_hbm, s_hbm, z_hbm, ms_hbm, md_hbm,
                  idxv, vb0, vb1, acc, sem0, sem1):
    c = lax.axis_index("c")
    s = lax.axis_index("s")
    col = c * HC
    base = s * SCH
    nch = jnp.where(s < NS - 1, SCH, SLAST)
    nbase = s * NPT

    def fire(q, vb, sem):
        pltpu.async_copy(e_hbm.at[pl.ds(q * GC, GC), pl.ds(col, HC)],
                         vb, sem)

    def wait(vb, sem):
        pltpu.make_async_copy(e_hbm.at[pl.ds(0, GC), pl.ds(col, HC)],
                              vb, sem).wait()

    for i_hbm, o_hbm in ((d_hbm, ms_hbm), (s_hbm, md_hbm)):
        @pl.when(s < NS - 1)
        def _():
            pltpu.sync_copy(z_hbm, acc.at[pl.ds(nbase, NPT)])

        @pl.when(s == NS - 1)
        def _():
            pltpu.sync_copy(z_hbm.at[pl.ds(0, NPTL)],
                            acc.at[pl.ds(nbase, NPTL)])

        plsc.subcore_barrier()
        fire(base, vb0, sem0)

        def blk(b, carry):
            pltpu.sync_copy(i_hbm.at[pl.ds(base + b * SB, SB)], idxv)
            nin = jnp.minimum(SB, nch - b * SB)

            def pair(pp, carry2):
                jj0 = 2 * pp
                q0 = base + b * SB + jj0
                fire(q0 + 1, vb1, sem1)
                wait(vb0, sem0)
                pltpu.sync_copy(vb0, acc.at[idxv.at[jj0]], add=True)

                @pl.when(q0 + 2 < base + nch)
                def _():
                    fire(q0 + 2, vb0, sem0)

                wait(vb1, sem1)
                pltpu.sync_copy(vb1, acc.at[idxv.at[jj0 + 1]], add=True)
                return carry2

            lax.fori_loop(0, nin // 2, pair, 0)
            return carry

        lax.fori_loop(0, (SCH + SB - 1) // SB, blk, 0)
        plsc.subcore_barrier()

        @pl.when(s < NS - 1)
        def _():
            pltpu.sync_copy(acc.at[pl.ds(nbase, NPT)],
                            o_hbm.at[pl.ds(nbase, NPT), pl.ds(col, HC)])

        @pl.when(s == NS - 1)
        def _():
            pltpu.sync_copy(acc.at[pl.ds(nbase, NPTL)],
                            o_hbm.at[pl.ds(nbase, NPTL), pl.ds(col, HC)])

        plsc.subcore_barrier()


def _make_scatter():
    mesh = plsc.VectorSubcoreMesh(core_axis_name="c", subcore_axis_name="s",
                                  num_cores=NC, num_subcores=NS)
    return pl.kernel(
        _scatter_body,
        out_type=[jax.ShapeDtypeStruct((N, H), jnp.float32),
                  jax.ShapeDtypeStruct((N, H), jnp.float32)],
        mesh=mesh,
        compiler_params=pltpu.CompilerParams(use_tc_tiling_on_sc=False),
        scratch_types=[
            pltpu.VMEM((SB, GC), jnp.int32),
            pltpu.VMEM((GC, HC), jnp.float32),
            pltpu.VMEM((GC, HC), jnp.float32),
            pltpu.VMEM_SHARED((N, HC), jnp.float32),
            pltpu.SemaphoreType.DMA,
            pltpu.SemaphoreType.DMA,
        ],
    )


def _w(p, i):
    return p[i]["W"]


def _b(p, i):
    return p[i]["b"].reshape(1, -1)


@jax.jit
def kernel(z, params, edge_index):
    ei = edge_index.astype(jnp.int32)
    src = ei[0]
    dst = ei[1]
    srcr = jnp.pad(src.reshape(NCHUNK, GC), ((0, CPAD - NCHUNK), (0, 0)))
    dstr = jnp.pad(dst.reshape(NCHUNK, GC), ((0, CPAD - NCHUNK), (0, 0)))
    zeros_tile = jnp.zeros((NPT, HC), jnp.float32)

    ne = params["node_encoder"]
    ee = params["edge_encoder"]
    en = params["edge_network"]
    nn = params["node_network"]
    dec = params["edge_decoder"]
    eot = params["edge_output_transform"]

    enW = [_w(en[i], 0) for i in range(ITERS)]
    nnW = [_w(nn[i], 0) for i in range(ITERS)]

    eeW1 = _w(ee, 0)
    wu0 = enW[0][2 * H:3 * H] + enW[0][3 * H:4 * H]
    wv0 = enW[0][4 * H:5 * H] + enW[0][5 * H:6 * H]

    enc = _tc_call(_enc_body, N, BN,
                   [1, (1, H), (1, H), (H, H), (1, H), (H, H), (H, H),
                    (H, H), (H, H)],
                   [H, H, H, H, H])
    x0, ue, ve, u0, v0 = enc(z.reshape(N, 1), _w(ne, 0), _b(ne, 0),
                             _w(ne, 1), _b(ne, 1), eeW1[:H], eeW1[H:],
                             wu0, wv0)

    gather = _make_gather()
    scatter = _make_scatter()

    edge0 = _tc_call(_edge0_body, E, BE,
                     [H, (1, H), (H, H), (1, H)], [H])
    edge_mlp = _tc_call(_edge_body, E, BE,
                        [H, H, H, (H, H), (H, H), (1, H), (H, H), (1, H)],
                        [H])
    node_tab = _tc_call(_node_tab_body, N, BN,
                        [H, H, H, H] + [(H, H)] * 4 + [(1, H), (H, H), (1, H)]
                        + [(H, H)] * 4,
                        [H, H, H])
    node_last = _tc_call(_node_body, N, BN,
                         [H, H, H, H] + [(H, H)] * 4
                         + [(1, H), (H, H), (1, H)],
                         [H])

    g = gather(ue, ve, srcr, dstr)
    e0 = edge0(g, _b(ee, 0), _w(ee, 1), _b(ee, 1))

    e = e0
    x = x0
    u, v = u0, v0
    for i in range(ITERS):
        g = gather(u, v, srcr, dstr)
        e = edge_mlp(e, e0, g, enW[i][:H], enW[i][H:2 * H], _b(en[i], 0),
                     _w(en[i], 1), _b(en[i], 1))
        ms, md = scatter(e, dstr, srcr, zeros_tile)
        if i < ITERS - 1:
            x, u, v = node_tab(ms, md, x, x0, nnW[i][:H], nnW[i][H:2 * H],
                               nnW[i][2 * H:3 * H], nnW[i][3 * H:],
                               _b(nn[i], 0), _w(nn[i], 1), _b(nn[i], 1),
                               enW[i + 1][2 * H:3 * H],
                               enW[i + 1][3 * H:4 * H],
                               enW[i + 1][4 * H:5 * H],
                               enW[i + 1][5 * H:6 * H])
        else:
            x = node_last(ms, md, x, x0, nnW[i][:H], nnW[i][H:2 * H],
                          nnW[i][2 * H:3 * H], nnW[i][3 * H:],
                          _b(nn[i], 0), _w(nn[i], 1), _b(nn[i], 1))

    decode = _tc_call(_decode_body, E, BE,
                      [H, (H, H), (1, H), (H, H), (1, H), (H, H), (1, H),
                       (1, H), (1, 1)],
                      [1])
    out = decode(e, _w(dec, 0), _b(dec, 0), _w(dec, 1), _b(dec, 1),
                 _w(eot, 0), _b(eot, 0), _w(eot, 1).reshape(1, H),
                 _b(eot, 1).reshape(1, 1))
    return out[:, 0]
